# baseline (speedup 1.0000x reference)
import functools

import jax
import jax.numpy as jnp
from jax import lax
from jax.experimental import pallas as pl
from jax.experimental.pallas import tpu as pltpu
from jax.experimental.pallas import tpu_sc as plsc

NC = 2
NS = 16
NW = NC * NS


def _attention(embsumm, mel, enc, mens, maskf, encsumm, mixer_W, mixer_b):
    B, MZ, D = mens.shape
    E = mel.shape[2]
    Bb = 16 if B % 16 == 0 else B

    def body(emb_r, mel_r, enc_r, mens_r, mask_r, encs_r, mw_r, mb_r,
             summ_o, vals_o, mix0_o):
        def _rb(x):
            return x.astype(jnp.bfloat16)

        bdot = functools.partial(
            lax.dot_general,
            dimension_numbers=(((2,), (1,)), ((0,), (0,))),
            preferred_element_type=jnp.float32)
        mens_v = mens_r[...]
        enc_v = enc_r[...]
        w = bdot(_rb(mens_v), _rb(enc_v))
        melb = _rb(mel_r[...]).astype(jnp.float32)
        embb = _rb(emb_r[...]).astype(jnp.float32)
        w = w + jnp.sum(melb * embb[:, None, :], axis=2)
        mask = mask_r[...]
        wm = jnp.where(mask > 0, w, -1e30)
        m = jnp.max(wm, axis=1, keepdims=True)
        e = jnp.where(mask > 0, jnp.exp(wm - m), 0.0)
        s = jnp.sum(e, axis=1, keepdims=True)
        alphas = e / s
        summ_o[...] = lax.dot_general(
            alphas, mens_v, (((1,), (1,)), ((0,), (0,))),
            precision=lax.Precision.HIGHEST,
            preferred_element_type=jnp.float32)
        h = jnp.concatenate([encs_r[...], enc_v], axis=1)
        l01 = lax.dot_general(
            h.astype(jnp.bfloat16), mw_r[...].astype(jnp.bfloat16),
            (((1,), (1,)), ((), ())),
            preferred_element_type=jnp.float32)
        l01 = l01 + mb_r[...][None, :]
        d = l01[:, 1] - l01[:, 0]
        mix0 = 1.0 / (1.0 + jnp.exp(d))
        mix1 = 1.0 / (1.0 + jnp.exp(-d))
        vals_o[...] = alphas * mix1[:, None]
        mix0_o[...] = jnp.broadcast_to(mix0[:, None], (Bb, 128))

    grid = (B // Bb,)
    return pl.pallas_call(
        body,
        grid=grid,
        in_specs=[
            pl.BlockSpec((Bb, E), lambda i: (i, 0)),
            pl.BlockSpec((Bb, MZ, E), lambda i: (i, 0, 0)),
            pl.BlockSpec((Bb, D), lambda i: (i, 0)),
            pl.BlockSpec((Bb, MZ, D), lambda i: (i, 0, 0)),
            pl.BlockSpec((Bb, MZ), lambda i: (i, 0)),
            pl.BlockSpec((Bb, D), lambda i: (i, 0)),
            pl.BlockSpec((2, 2 * D), lambda i: (0, 0)),
            pl.BlockSpec((2,), lambda i: (0,)),
        ],
        out_specs=[
            pl.BlockSpec((Bb, D), lambda i: (i, 0)),
            pl.BlockSpec((Bb, MZ), lambda i: (i, 0)),
            pl.BlockSpec((Bb, 128), lambda i: (i, 0)),
        ],
        out_shape=[
            jax.ShapeDtypeStruct((B, D), jnp.float32),
            jax.ShapeDtypeStruct((B, MZ), jnp.float32),
            jax.ShapeDtypeStruct((B, 128), jnp.float32),
        ],
    )(embsumm, mel, enc, mens, maskf, encsumm, mixer_W, mixer_b)


def _vocab_pass1(h, W, bias, umask, VB):
    B, D2 = h.shape
    V = W.shape[0]
    NB = -(-V // VB)

    def body(h_r, w_r, b_r, um_r, p_o, ms_o):
        nb = pl.program_id(0)
        l = lax.dot_general(
            h_r[...].astype(jnp.bfloat16), w_r[...].astype(jnp.bfloat16),
            (((1,), (1,)), ((), ())),
            preferred_element_type=jnp.float32)
        l = l + b_r[...][None, :]
        msk = um_r[...]
        col = nb * VB + lax.broadcasted_iota(jnp.int32, (1, VB), 1)
        valid = (col < V) & (msk[None, :] > 0)
        lm = jnp.where(valid, l, -1e30)
        m = jnp.max(lm, axis=1)
        e = jnp.where(valid, jnp.exp(lm - m[:, None]), 0.0)
        p_o[...] = e
        ms_o[0, 0, :] = m
        ms_o[0, 1, :] = jnp.sum(e, axis=1)

    return pl.pallas_call(
        body,
        grid=(NB,),
        in_specs=[
            pl.BlockSpec((B, D2), lambda nb: (0, 0)),
            pl.BlockSpec((VB, D2), lambda nb: (nb, 0)),
            pl.BlockSpec((VB,), lambda nb: (nb,)),
            pl.BlockSpec((VB,), lambda nb: (nb,)),
        ],
        out_specs=[
            pl.BlockSpec((B, VB), lambda nb: (0, nb)),
            pl.BlockSpec((1, 2, B), lambda nb: (nb, 0, 0)),
        ],
        out_shape=[
            jax.ShapeDtypeStruct((B, NB * VB), jnp.float32),
            jax.ShapeDtypeStruct((NB, 2, B), jnp.float32),
        ],
    )(h, W, bias, umask)


def _sc_combine(P, crep, ids, vals, V, VB):
    B, VP = P.shape
    NB = VP // VB
    MZ = ids.shape[1]
    VTAIL = V - (NB - 1) * VB
    RW = B // NW
    mesh = plsc.VectorSubcoreMesh(core_axis_name="c", subcore_axis_name="s")

    QS = [0, NB // 4, NB // 2, (3 * NB) // 4, NB]

    def body(p_hbm, crep_hbm, ids_hbm, vals_hbm, out_hbm,
             rowbuf, ids_v, vals_v, crep_v, probe,
             sem0, sem1, sem2, sem3):
        wid = lax.axis_index("c") * NS + lax.axis_index("s")
        lane = lax.iota(jnp.int32, 16)
        sems = [sem0, sem1, sem2, sem3]

        def row_body(r, c):
            b = wid * RW + r
            handles = []
            for q in range(4):
                lo, hi = QS[q] * VB, QS[q + 1] * VB
                handles.append(pltpu.async_copy(
                    p_hbm.at[b, pl.ds(lo, hi - lo)],
                    rowbuf.at[pl.ds(lo, hi - lo)], sems[q]))
            pltpu.sync_copy(ids_hbm.at[b], ids_v)
            pltpu.sync_copy(vals_hbm.at[b], vals_v)
            pltpu.sync_copy(crep_hbm.at[b], crep_v)

            def blk(nb, c2):
                cvec = crep_v[pl.ds(nb * 16, 16)]
                base = nb * VB

                @plsc.parallel_loop(0, VB // 16, 1, unroll=8)
                def _scale(i):
                    o = base + i * 16
                    rowbuf[pl.ds(o, 16)] = rowbuf[pl.ds(o, 16)] * cvec
                return c2
            for q in range(4):
                handles[q].wait()
                lax.fori_loop(QS[q], QS[q + 1], blk, 0)

            def scat(j, c2):
                k = ids_v[pl.ds(j * 16, 16)]
                v = vals_v[pl.ds(j * 16, 16)]
                hk = jnp.bitwise_and(k, 1023)
                plsc.store_scatter(probe, [hk], lane)
                rb = plsc.load_gather(probe, [hk])
                cnt = plsc.all_reduce_population_count(rb == lane)
                nodup = jnp.max(cnt) == 16

                @pl.when(nodup)
                def _fast():
                    plsc.addupdate_scatter(rowbuf, [k], v)

                @pl.when(jnp.logical_not(nodup))
                def _slow():
                    for t in range(16):
                        plsc.addupdate_scatter(rowbuf, [k], v, mask=lane == t)
                return c2
            lax.fori_loop(0, MZ // 16, scat, 0)

            pltpu.sync_copy(rowbuf.at[pl.ds(0, V)],
                            out_hbm.at[pl.ds(b * V, V)])
            return c
        lax.fori_loop(0, RW, row_body, 0)

    f = pl.kernel(
        body,
        out_type=jax.ShapeDtypeStruct((B * V,), jnp.float32),
        mesh=mesh,
        compiler_params=pltpu.CompilerParams(needs_layout_passes=False),
        scratch_types=[
            pltpu.VMEM((VP,), jnp.float32),
            pltpu.VMEM((MZ,), jnp.int32),
            pltpu.VMEM((MZ,), jnp.float32),
            pltpu.VMEM((NB * 16,), jnp.float32),
            pltpu.VMEM((1024,), jnp.int32),
            pltpu.SemaphoreType.DMA,
            pltpu.SemaphoreType.DMA,
            pltpu.SemaphoreType.DMA,
            pltpu.SemaphoreType.DMA,
        ],
    )
    return f(P, crep, ids, vals)


def kernel(enc, encsumm, embsumm, memencs, memencsumm, memembsumm, memmask,
           memids, outlin_W, outlin_b, mixer_W, mixer_b, unktok_mask):
    B, D = enc.shape
    M, Z = memids.shape[1], memids.shape[2]
    MZ = M * Z
    V = unktok_mask.shape[0]
    VB = 2048

    mel = memembsumm.reshape(B, MZ, memembsumm.shape[3])
    mens = memencs.reshape(B, MZ, D)
    maskf = memmask.reshape(B, MZ)

    summ, vals, mix0r = _attention(
        embsumm, mel, enc, mens, maskf, encsumm, mixer_W, mixer_b)

    h = jnp.concatenate([encsumm, enc], axis=-1)
    P, ms = _vocab_pass1(h, outlin_W, outlin_b, unktok_mask, VB)

    m_loc = ms[:, 0, :].T
    s_loc = ms[:, 1, :].T
    gmax = jnp.max(m_loc, axis=1, keepdims=True)
    r = jnp.exp(m_loc - gmax)
    denom = jnp.sum(r * s_loc, axis=1, keepdims=True)
    coeff = r / denom * mix0r[:, :1]
    crep = jnp.repeat(coeff, 16, axis=1)

    ids = memids.reshape(B, MZ)
    probs = _sc_combine(P, crep, ids, vals, V, VB).reshape(B, V)
    return (probs, summ)


# speedup vs baseline: 1.1250x; 1.1250x over previous
import functools

import jax
import jax.numpy as jnp
from jax import lax
from jax.experimental import pallas as pl
from jax.experimental.pallas import tpu as pltpu
from jax.experimental.pallas import tpu_sc as plsc

NC = 2
NS = 16
NW = NC * NS


def _attention(embsumm, mel, enc, mens, maskf, encsumm, mixer_W, mixer_b):
    B, MZ, D = mens.shape
    E = mel.shape[2]
    Bb = 16 if B % 16 == 0 else B

    def body(emb_r, mel_r, enc_r, mens_r, mask_r, encs_r, mw_r, mb_r,
             summ_o, vals_o, mix0_o):
        def _rb(x):
            return x.astype(jnp.bfloat16)

        bdot = functools.partial(
            lax.dot_general,
            dimension_numbers=(((2,), (1,)), ((0,), (0,))),
            preferred_element_type=jnp.float32)
        mens_v = mens_r[...]
        enc_v = enc_r[...]
        w = bdot(_rb(mens_v), _rb(enc_v))
        melb = _rb(mel_r[...]).astype(jnp.float32)
        embb = _rb(emb_r[...]).astype(jnp.float32)
        w = w + jnp.sum(melb * embb[:, None, :], axis=2)
        mask = mask_r[...]
        wm = jnp.where(mask > 0, w, -1e30)
        m = jnp.max(wm, axis=1, keepdims=True)
        e = jnp.where(mask > 0, jnp.exp(wm - m), 0.0)
        s = jnp.sum(e, axis=1, keepdims=True)
        alphas = e / s
        summ_o[...] = lax.dot_general(
            alphas, mens_v, (((1,), (1,)), ((0,), (0,))),
            precision=lax.Precision.HIGHEST,
            preferred_element_type=jnp.float32)
        h = jnp.concatenate([encs_r[...], enc_v], axis=1)
        l01 = lax.dot_general(
            h.astype(jnp.bfloat16), mw_r[...].astype(jnp.bfloat16),
            (((1,), (1,)), ((), ())),
            preferred_element_type=jnp.float32)
        l01 = l01 + mb_r[...][None, :]
        d = l01[:, 1] - l01[:, 0]
        mix0 = 1.0 / (1.0 + jnp.exp(d))
        mix1 = 1.0 / (1.0 + jnp.exp(-d))
        vals_o[...] = alphas * mix1[:, None]
        mix0_o[...] = jnp.broadcast_to(mix0[:, None], (Bb, 128))

    grid = (B // Bb,)
    return pl.pallas_call(
        body,
        grid=grid,
        in_specs=[
            pl.BlockSpec((Bb, E), lambda i: (i, 0)),
            pl.BlockSpec((Bb, MZ, E), lambda i: (i, 0, 0)),
            pl.BlockSpec((Bb, D), lambda i: (i, 0)),
            pl.BlockSpec((Bb, MZ, D), lambda i: (i, 0, 0)),
            pl.BlockSpec((Bb, MZ), lambda i: (i, 0)),
            pl.BlockSpec((Bb, D), lambda i: (i, 0)),
            pl.BlockSpec((2, 2 * D), lambda i: (0, 0)),
            pl.BlockSpec((2,), lambda i: (0,)),
        ],
        out_specs=[
            pl.BlockSpec((Bb, D), lambda i: (i, 0)),
            pl.BlockSpec((Bb, MZ), lambda i: (i, 0)),
            pl.BlockSpec((Bb, 128), lambda i: (i, 0)),
        ],
        out_shape=[
            jax.ShapeDtypeStruct((B, D), jnp.float32),
            jax.ShapeDtypeStruct((B, MZ), jnp.float32),
            jax.ShapeDtypeStruct((B, 128), jnp.float32),
        ],
    )(embsumm, mel, enc, mens, maskf, encsumm, mixer_W, mixer_b)


def _vocab_pass1(h, W, bias, umask, VB):
    B, D2 = h.shape
    V = W.shape[0]
    NB = -(-V // VB)

    def body(h_r, w_r, b_r, um_r, p_o, ms_o):
        nb = pl.program_id(0)
        l = lax.dot_general(
            h_r[...].astype(jnp.bfloat16), w_r[...].astype(jnp.bfloat16),
            (((1,), (1,)), ((), ())),
            preferred_element_type=jnp.float32)
        l = l + b_r[...][None, :]
        msk = um_r[...]
        col = nb * VB + lax.broadcasted_iota(jnp.int32, (1, VB), 1)
        valid = (col < V) & (msk[None, :] > 0)
        lm = jnp.where(valid, l, -1e30)
        m = jnp.max(lm, axis=1)
        e = jnp.where(valid, jnp.exp(lm - m[:, None]), 0.0)
        p_o[...] = e
        ms_o[0, 0, :] = m
        ms_o[0, 1, :] = jnp.sum(e, axis=1)

    return pl.pallas_call(
        body,
        grid=(NB,),
        in_specs=[
            pl.BlockSpec((B, D2), lambda nb: (0, 0)),
            pl.BlockSpec((VB, D2), lambda nb: (nb, 0)),
            pl.BlockSpec((VB,), lambda nb: (nb,)),
            pl.BlockSpec((VB,), lambda nb: (nb,)),
        ],
        out_specs=[
            pl.BlockSpec((B, VB), lambda nb: (0, nb)),
            pl.BlockSpec((1, 2, B), lambda nb: (nb, 0, 0)),
        ],
        out_shape=[
            jax.ShapeDtypeStruct((B, NB * VB), jnp.float32),
            jax.ShapeDtypeStruct((NB, 2, B), jnp.float32),
        ],
    )(h, W, bias, umask)


def _sc_combine(P, crep, ids, vals, V, VB):
    B, VP = P.shape
    NB = VP // VB
    MZ = ids.shape[1]
    VTAIL = V - (NB - 1) * VB
    RW = B // NW
    mesh = plsc.VectorSubcoreMesh(core_axis_name="c", subcore_axis_name="s")
    Pf = P.reshape(B * VP)

    NBF = NB - 1
    QS = [0, NBF // 4, NBF // 2, (3 * NBF) // 4, NBF]
    QW = [QS[0] * VB, QS[1] * VB, QS[2] * VB, QS[3] * VB, V]

    def body(p_hbm, crep_hbm, ids_hbm, vals_hbm, out_hbm,
             rowbuf, ids_v, vals_v, crep_v, probe,
             sem0, sem1, sem2, sem3):
        wid = lax.axis_index("c") * NS + lax.axis_index("s")
        lane = lax.iota(jnp.int32, 16)
        sems = [sem0, sem1, sem2, sem3]

        def row_body(r, c):
            b = wid * RW + r
            handles = []
            for q in range(4):
                lo, hi = QW[q], QW[q + 1]
                handles.append(pltpu.async_copy(
                    p_hbm.at[pl.ds(b * VP + lo, hi - lo)],
                    rowbuf.at[pl.ds(lo, hi - lo)], sems[q]))
            pltpu.sync_copy(ids_hbm.at[b], ids_v)
            pltpu.sync_copy(vals_hbm.at[b], vals_v)
            pltpu.sync_copy(crep_hbm.at[b], crep_v)

            def blk(nb, c2):
                cvec = crep_v[pl.ds(nb * 16, 16)]
                base = nb * VB

                @plsc.parallel_loop(0, VB // 16, 1, unroll=8)
                def _scale(i):
                    o = base + i * 16
                    rowbuf[pl.ds(o, 16)] = rowbuf[pl.ds(o, 16)] * cvec
                return c2
            for q in range(4):
                handles[q].wait()
                lax.fori_loop(QS[q], QS[q + 1], blk, 0)
            tvec = crep_v[pl.ds(NBF * 16, 16)]

            @plsc.parallel_loop(0, VTAIL // 16, 1, unroll=8)
            def _scale_tail(i):
                o = NBF * VB + i * 16
                rowbuf[pl.ds(o, 16)] = rowbuf[pl.ds(o, 16)] * tvec

            def scat(j, c2):
                k = ids_v[pl.ds(j * 16, 16)]
                v = vals_v[pl.ds(j * 16, 16)]
                hk = jnp.bitwise_and(k, 1023)
                plsc.store_scatter(probe, [hk], lane)
                rb = plsc.load_gather(probe, [hk])
                cnt = plsc.all_reduce_population_count(rb == lane)
                nodup = jnp.max(cnt) == 16

                @pl.when(nodup)
                def _fast():
                    plsc.addupdate_scatter(rowbuf, [k], v)

                @pl.when(jnp.logical_not(nodup))
                def _slow():
                    for t in range(16):
                        plsc.addupdate_scatter(rowbuf, [k], v, mask=lane == t)
                return c2
            lax.fori_loop(0, MZ // 16, scat, 0)

            pltpu.sync_copy(rowbuf, out_hbm.at[b])
            return c
        lax.fori_loop(0, RW, row_body, 0)

    f = pl.kernel(
        body,
        out_type=jax.ShapeDtypeStruct((B, V), jnp.float32),
        mesh=mesh,
        compiler_params=pltpu.CompilerParams(needs_layout_passes=False),
        scratch_types=[
            pltpu.VMEM((V,), jnp.float32),
            pltpu.VMEM((MZ,), jnp.int32),
            pltpu.VMEM((MZ,), jnp.float32),
            pltpu.VMEM((NB * 16,), jnp.float32),
            pltpu.VMEM((1024,), jnp.int32),
            pltpu.SemaphoreType.DMA,
            pltpu.SemaphoreType.DMA,
            pltpu.SemaphoreType.DMA,
            pltpu.SemaphoreType.DMA,
        ],
    )
    return f(Pf, crep, ids, vals)


def kernel(enc, encsumm, embsumm, memencs, memencsumm, memembsumm, memmask,
           memids, outlin_W, outlin_b, mixer_W, mixer_b, unktok_mask):
    B, D = enc.shape
    M, Z = memids.shape[1], memids.shape[2]
    MZ = M * Z
    V = unktok_mask.shape[0]
    VB = 2048

    mel = memembsumm.reshape(B, MZ, memembsumm.shape[3])
    mens = memencs.reshape(B, MZ, D)
    maskf = memmask.reshape(B, MZ)

    summ, vals, mix0r = _attention(
        embsumm, mel, enc, mens, maskf, encsumm, mixer_W, mixer_b)

    h = jnp.concatenate([encsumm, enc], axis=-1)
    P, ms = _vocab_pass1(h, outlin_W, outlin_b, unktok_mask, VB)

    m_loc = ms[:, 0, :].T
    s_loc = ms[:, 1, :].T
    gmax = jnp.max(m_loc, axis=1, keepdims=True)
    r = jnp.exp(m_loc - gmax)
    denom = jnp.sum(r * s_loc, axis=1, keepdims=True)
    coeff = r / denom * mix0r[:, :1]
    crep = jnp.repeat(coeff, 16, axis=1)

    ids = memids.reshape(B, MZ)
    probs = _sc_combine(P, crep, ids, vals, V, VB)
    return (probs, summ)
